# Initial kernel scaffold; baseline (speedup 1.0000x reference)
import jax
import jax.numpy as jnp
from jax.experimental import pallas as pl


def kernel(x, edge_index, W1, b1, W2, b2, temp):
    raise NotImplementedError("write your pallas kernel here")


# speedup vs baseline: 4.5102x; 4.5102x over previous
import numpy as np
from math import comb

import jax
import jax.numpy as jnp
from jax.experimental import pallas as pl

_K = 10
_N = 10000

_GW = np.zeros((_K + 1, _K + 1), np.float32)
for _m in range(_K + 1):
    for _j in range(_K + 1):
        _s = 0
        for _a in range(max(0, _j - (_K - _m)), min(_m, _j) + 1):
            _s += comb(_m, _a) * (-1) ** _a * comb(_K - _m, _j - _a)

        _GW[_m, _j] = comb(_K, _m) * _s
_GW_J = jnp.asarray(_GW)


def kernel(x, edge_index, W1, b1, W2, b2, temp):
    t = jax.nn.relu(temp)
    alpha = jnp.sum(t[:, None] * _GW_J, axis=0) * (1.0 / 2 ** _K)

    h = jax.nn.relu(x @ W1.T + b1)
    h = h @ W2.T + b2

    src = edge_index[0]
    dst = edge_index[1]
    deg = jnp.zeros((_N,), jnp.float32).at[src].add(1.0)
    d1 = jnp.where(deg > 0, jax.lax.rsqrt(deg), 0.0)[:, None]
    d2 = d1 * d1
    g = d1 * h

    u = alpha[_K] * g
    for j in range(_K - 1, 0, -1):
        agg = jnp.zeros_like(u).at[dst].add(u[src])
        u = d2 * agg + alpha[j] * g
    agg = jnp.zeros_like(u).at[dst].add(u[src])
    return d1 * agg + alpha[0] * h


# speedup vs baseline: 9.1414x; 2.0268x over previous
import functools
from math import comb

import numpy as np
import jax
import jax.numpy as jnp
from jax import lax
from jax.experimental import pallas as pl
from jax.experimental.pallas import tpu as pltpu
from jax.experimental.pallas import tpu_sc as plsc

_K = 10
_N = 10000
_E = 160000
_D = 128

_NPAD = 12032
_GARBAGE = _N
_CHUNK = 128
_NTILES = 32
_CPT = 40
_EPAD = _NTILES * _CPT * _CHUNK
_ROWS_PT = _NPAD // 16
_BLK = 1000
_GRID = _N // _BLK

_GW = np.zeros((_K + 1, _K + 1), np.float32)
for _m in range(_K + 1):
    for _j in range(_K + 1):
        _s = 0
        for _a in range(max(0, _j - (_K - _m)), min(_m, _j) + 1):
            _s += comb(_m, _a) * (-1) ** _a * comb(_K - _m, _j - _a)

        _GW[_m, _j] = comb(_K, _m) * _s


_sc_mesh = plsc.VectorSubcoreMesh(core_axis_name="c", subcore_axis_name="s")


@functools.partial(
    pl.kernel,
    out_type=jax.ShapeDtypeStruct((2, _NPAD, _D), jnp.float32),
    mesh=_sc_mesh,
    scratch_types=[
        pltpu.VMEM((_CPT, _CHUNK), jnp.int32),
        pltpu.VMEM((_CPT, _CHUNK), jnp.int32),
        pltpu.VMEM((_CHUNK, _D), jnp.float32),
        pltpu.VMEM_SHARED((_NPAD, _D), jnp.float32),
        pltpu.SemaphoreType.DMA,
    ],
)
def _sc_gather_scatter(table_hbm, srcg_hbm, dsts_hbm, zeros_hbm, out_hbm,
                       src_v, dst_v, buf_v, acc, sem):
    c = lax.axis_index("c")
    s = lax.axis_index("s")
    wid = c * 16 + s
    pltpu.sync_copy(zeros_hbm.at[pl.ds(s * _ROWS_PT, _ROWS_PT)],
                    acc.at[pl.ds(s * _ROWS_PT, _ROWS_PT)])
    pltpu.sync_copy(srcg_hbm.at[pl.ds(wid * _CPT, _CPT)], src_v)
    pltpu.sync_copy(dsts_hbm.at[pl.ds(wid * _CPT, _CPT)], dst_v)
    plsc.subcore_barrier()

    def body(i, carry):
        pltpu.async_copy(table_hbm.at[src_v.at[i]], buf_v, sem).wait()
        pltpu.sync_copy(buf_v, acc.at[dst_v.at[i]], add=True)
        return carry

    lax.fori_loop(0, _CPT, body, 0)
    plsc.subcore_barrier()
    pltpu.sync_copy(acc.at[pl.ds(s * _ROWS_PT, _ROWS_PT)],
                    out_hbm.at[c, pl.ds(s * _ROWS_PT, _ROWS_PT)])


def _mlp_body(x_ref, w1t_ref, b1_ref, w2t_ref, b2_ref, o_ref):
    h1 = jnp.dot(x_ref[...], w1t_ref[...],
                 preferred_element_type=jnp.float32,
                 precision=lax.Precision.HIGHEST) + b1_ref[...]
    h1 = jnp.maximum(h1, 0.0)
    o_ref[...] = jnp.dot(h1, w2t_ref[...],
                         preferred_element_type=jnp.float32,
                         precision=lax.Precision.HIGHEST) + b2_ref[...]


def _mlp(x, w1t, b1, w2t, b2):
    return pl.pallas_call(
        _mlp_body,
        grid=(_GRID,),
        in_specs=[
            pl.BlockSpec((_BLK, _D), lambda i: (i, 0)),
            pl.BlockSpec((_D, _D), lambda i: (0, 0)),
            pl.BlockSpec((1, _D), lambda i: (0, 0)),
            pl.BlockSpec((_D, _D), lambda i: (0, 0)),
            pl.BlockSpec((1, _D), lambda i: (0, 0)),
        ],
        out_specs=pl.BlockSpec((_BLK, _D), lambda i: (i, 0)),
        out_shape=jax.ShapeDtypeStruct((_N, _D), jnp.float32),
    )(x, w1t, b1, w2t, b2)


def _prep_body(acc_ref, h_ref, alpha_ref, d1_ref, d2_ref, g_ref, u_ref):
    deg = acc_ref[0, :, 0:1] + acc_ref[1, :, 0:1]
    d1 = jnp.where(deg > 0.0, lax.rsqrt(deg), 0.0)
    d1_ref[...] = d1
    d2_ref[...] = d1 * d1
    g = d1 * h_ref[...]
    g_ref[...] = g
    u_ref[...] = alpha_ref[0, 0] * g


def _prep(degacc, h, alpha_k):
    return pl.pallas_call(
        _prep_body,
        grid=(_GRID,),
        in_specs=[
            pl.BlockSpec((2, _BLK, _D), lambda i: (0, i, 0)),
            pl.BlockSpec((_BLK, _D), lambda i: (i, 0)),
            pl.BlockSpec(memory_space=pltpu.SMEM),
        ],
        out_specs=[
            pl.BlockSpec((_BLK, 1), lambda i: (i, 0)),
            pl.BlockSpec((_BLK, 1), lambda i: (i, 0)),
            pl.BlockSpec((_BLK, _D), lambda i: (i, 0)),
            pl.BlockSpec((_BLK, _D), lambda i: (i, 0)),
        ],
        out_shape=[
            jax.ShapeDtypeStruct((_N, 1), jnp.float32),
            jax.ShapeDtypeStruct((_N, 1), jnp.float32),
            jax.ShapeDtypeStruct((_N, _D), jnp.float32),
            jax.ShapeDtypeStruct((_N, _D), jnp.float32),
        ],
    )(degacc, h, alpha_k)


def _combine_body(scale_ref, base_ref, acc_ref, alpha_ref, o_ref):
    ssum = acc_ref[0] + acc_ref[1]
    o_ref[...] = scale_ref[...] * ssum + alpha_ref[0, 0] * base_ref[...]


def _combine(scale, base, sacc, alpha_j):
    return pl.pallas_call(
        _combine_body,
        grid=(_GRID,),
        in_specs=[
            pl.BlockSpec((_BLK, 1), lambda i: (i, 0)),
            pl.BlockSpec((_BLK, _D), lambda i: (i, 0)),
            pl.BlockSpec((2, _BLK, _D), lambda i: (0, i, 0)),
            pl.BlockSpec(memory_space=pltpu.SMEM),
        ],
        out_specs=pl.BlockSpec((_BLK, _D), lambda i: (i, 0)),
        out_shape=jax.ShapeDtypeStruct((_N, _D), jnp.float32),
    )(scale, base, sacc, alpha_j)


def kernel(x, edge_index, W1, b1, W2, b2, temp):
    t = jax.nn.relu(temp)
    alpha = jnp.sum(t[:, None] * jnp.asarray(_GW), axis=0) * (1.0 / 2 ** _K)

    h = _mlp(x, W1.T, b1.reshape(1, _D), W2.T, b2.reshape(1, _D))

    src = edge_index[0]
    dst = edge_index[1]
    npad_e = _EPAD - _E
    pad_g = jnp.zeros((npad_e,), jnp.int32)
    pad_s = jnp.full((npad_e,), _GARBAGE, jnp.int32)
    srcg = jnp.concatenate([src, pad_g]).reshape(_EPAD // _CHUNK, _CHUNK)
    dsts = jnp.concatenate([dst, pad_s]).reshape(_EPAD // _CHUNK, _CHUNK)
    srcs = jnp.concatenate([src, pad_s]).reshape(_EPAD // _CHUNK, _CHUNK)

    zeros = jnp.zeros((_NPAD, _D), jnp.float32)
    ones_tab = jnp.ones((_N, _D), jnp.float32)

    degacc = _sc_gather_scatter(ones_tab, srcg, srcs, zeros)
    d1, d2, g, u = _prep(degacc, h, alpha[_K].reshape(1, 1))

    for j in range(_K - 1, 0, -1):
        sacc = _sc_gather_scatter(u, srcg, dsts, zeros)
        u = _combine(d2, g, sacc, alpha[j].reshape(1, 1))
    sacc = _sc_gather_scatter(u, srcg, dsts, zeros)
    return _combine(d1, h, sacc, alpha[0].reshape(1, 1))


# speedup vs baseline: 11.6318x; 1.2724x over previous
import functools
from math import comb

import numpy as np
import jax
import jax.numpy as jnp
from jax import lax
from jax.experimental import pallas as pl
from jax.experimental.pallas import tpu as pltpu
from jax.experimental.pallas import tpu_sc as plsc

_K = 10
_N = 10000
_E = 160000
_D = 128

_NPAD = 10112
_GARBAGE = _N
_CHUNK = 128
_NTILES = 32
_CPT = 40
_NBUF = 2
_EPAD = _NTILES * _CPT * _CHUNK
_ROWS_PT = _NPAD // 16
_BLK = 1000
_GRID = _N // _BLK

_GW = np.zeros((_K + 1, _K + 1), np.float32)
for _m in range(_K + 1):
    for _j in range(_K + 1):
        _s = 0
        for _a in range(max(0, _j - (_K - _m)), min(_m, _j) + 1):
            _s += comb(_m, _a) * (-1) ** _a * comb(_K - _m, _j - _a)

        _GW[_m, _j] = comb(_K, _m) * _s


_sc_mesh = plsc.VectorSubcoreMesh(core_axis_name="c", subcore_axis_name="s")


@functools.partial(
    pl.kernel,
    out_type=jax.ShapeDtypeStruct((2, _NPAD, _D), jnp.float32),
    mesh=_sc_mesh,
    scratch_types=[
        pltpu.VMEM((_CPT, _CHUNK), jnp.int32),
        pltpu.VMEM((_CPT, _CHUNK), jnp.int32),
        [pltpu.VMEM((_CHUNK, _D), jnp.float32) for _ in range(_NBUF)],
        pltpu.VMEM_SHARED((_NPAD, _D), jnp.float32),
        [pltpu.SemaphoreType.DMA for _ in range(_NBUF)],
    ],
)
def _sc_gather_scatter(table_hbm, srcg_hbm, dsts_hbm, zeros_hbm, out_hbm,
                       src_v, dst_v, bufs, acc, sems):
    c = lax.axis_index("c")
    s = lax.axis_index("s")
    wid = c * 16 + s
    pltpu.sync_copy(zeros_hbm.at[pl.ds(s * _ROWS_PT, _ROWS_PT)],
                    acc.at[pl.ds(s * _ROWS_PT, _ROWS_PT)])
    pltpu.sync_copy(srcg_hbm.at[pl.ds(wid * _CPT, _CPT)], src_v)
    pltpu.sync_copy(dsts_hbm.at[pl.ds(wid * _CPT, _CPT)], dst_v)
    plsc.subcore_barrier()

    for b in range(_NBUF):
        pltpu.async_copy(table_hbm.at[src_v.at[b]], bufs[b], sems[b])

    def body(io, carry):
        i0 = io * _NBUF
        for b in range(_NBUF):
            i = i0 + b
            pltpu.make_async_copy(table_hbm.at[src_v.at[i]],
                                  bufs[b], sems[b]).wait()
            pltpu.sync_copy(bufs[b], acc.at[dst_v.at[i]], add=True)
            pltpu.async_copy(table_hbm.at[src_v.at[i + _NBUF]],
                             bufs[b], sems[b])
        return carry

    lax.fori_loop(0, (_CPT - _NBUF) // _NBUF, body, 0)
    for b in range(_NBUF):
        i = _CPT - _NBUF + b
        pltpu.make_async_copy(table_hbm.at[src_v.at[i]],
                              bufs[b], sems[b]).wait()
        pltpu.sync_copy(bufs[b], acc.at[dst_v.at[i]], add=True)
    plsc.subcore_barrier()
    pltpu.sync_copy(acc.at[pl.ds(s * _ROWS_PT, _ROWS_PT)],
                    out_hbm.at[c, pl.ds(s * _ROWS_PT, _ROWS_PT)])


def _mlp_body(x_ref, w1t_ref, b1_ref, w2t_ref, b2_ref, o_ref):
    h1 = jnp.dot(x_ref[...], w1t_ref[...],
                 preferred_element_type=jnp.float32,
                 precision=lax.Precision.HIGHEST) + b1_ref[...]
    h1 = jnp.maximum(h1, 0.0)
    o_ref[...] = jnp.dot(h1, w2t_ref[...],
                         preferred_element_type=jnp.float32,
                         precision=lax.Precision.HIGHEST) + b2_ref[...]


def _mlp(x, w1t, b1, w2t, b2):
    return pl.pallas_call(
        _mlp_body,
        grid=(_GRID,),
        in_specs=[
            pl.BlockSpec((_BLK, _D), lambda i: (i, 0)),
            pl.BlockSpec((_D, _D), lambda i: (0, 0)),
            pl.BlockSpec((1, _D), lambda i: (0, 0)),
            pl.BlockSpec((_D, _D), lambda i: (0, 0)),
            pl.BlockSpec((1, _D), lambda i: (0, 0)),
        ],
        out_specs=pl.BlockSpec((_BLK, _D), lambda i: (i, 0)),
        out_shape=jax.ShapeDtypeStruct((_N, _D), jnp.float32),
    )(x, w1t, b1, w2t, b2)


def _prep_body(acc_ref, h_ref, alpha_ref, d1_ref, d2_ref, g_ref, u_ref):
    deg = acc_ref[0, :, 0:1] + acc_ref[1, :, 0:1]
    d1 = jnp.where(deg > 0.0, lax.rsqrt(deg), 0.0)
    d1_ref[...] = d1
    d2_ref[...] = d1 * d1
    g = d1 * h_ref[...]
    g_ref[...] = g
    u_ref[...] = alpha_ref[0, 0] * g


def _prep(degacc, h, alpha_k):
    return pl.pallas_call(
        _prep_body,
        grid=(_GRID,),
        in_specs=[
            pl.BlockSpec((2, _BLK, _D), lambda i: (0, i, 0)),
            pl.BlockSpec((_BLK, _D), lambda i: (i, 0)),
            pl.BlockSpec(memory_space=pltpu.SMEM),
        ],
        out_specs=[
            pl.BlockSpec((_BLK, 1), lambda i: (i, 0)),
            pl.BlockSpec((_BLK, 1), lambda i: (i, 0)),
            pl.BlockSpec((_BLK, _D), lambda i: (i, 0)),
            pl.BlockSpec((_BLK, _D), lambda i: (i, 0)),
        ],
        out_shape=[
            jax.ShapeDtypeStruct((_N, 1), jnp.float32),
            jax.ShapeDtypeStruct((_N, 1), jnp.float32),
            jax.ShapeDtypeStruct((_N, _D), jnp.float32),
            jax.ShapeDtypeStruct((_N, _D), jnp.float32),
        ],
    )(degacc, h, alpha_k)


def _combine_body(scale_ref, base_ref, acc_ref, alpha_ref, o_ref):
    ssum = acc_ref[0] + acc_ref[1]
    o_ref[...] = scale_ref[...] * ssum + alpha_ref[0, 0] * base_ref[...]


def _combine(scale, base, sacc, alpha_j):
    return pl.pallas_call(
        _combine_body,
        grid=(_GRID,),
        in_specs=[
            pl.BlockSpec((_BLK, 1), lambda i: (i, 0)),
            pl.BlockSpec((_BLK, _D), lambda i: (i, 0)),
            pl.BlockSpec((2, _BLK, _D), lambda i: (0, i, 0)),
            pl.BlockSpec(memory_space=pltpu.SMEM),
        ],
        out_specs=pl.BlockSpec((_BLK, _D), lambda i: (i, 0)),
        out_shape=jax.ShapeDtypeStruct((_N, _D), jnp.float32),
    )(scale, base, sacc, alpha_j)


def kernel(x, edge_index, W1, b1, W2, b2, temp):
    t = jax.nn.relu(temp)
    alpha = jnp.sum(t[:, None] * jnp.asarray(_GW), axis=0) * (1.0 / 2 ** _K)

    h = _mlp(x, W1.T, b1.reshape(1, _D), W2.T, b2.reshape(1, _D))

    src = edge_index[0]
    dst = edge_index[1]
    npad_e = _EPAD - _E
    pad_g = jnp.zeros((npad_e,), jnp.int32)
    pad_s = jnp.full((npad_e,), _GARBAGE, jnp.int32)
    srcg = jnp.concatenate([src, pad_g]).reshape(_EPAD // _CHUNK, _CHUNK)
    dsts = jnp.concatenate([dst, pad_s]).reshape(_EPAD // _CHUNK, _CHUNK)
    srcs = jnp.concatenate([src, pad_s]).reshape(_EPAD // _CHUNK, _CHUNK)

    zeros = jnp.zeros((_NPAD, _D), jnp.float32)
    ones_tab = jnp.ones((_N, _D), jnp.float32)

    degacc = _sc_gather_scatter(ones_tab, srcg, srcs, zeros)
    d1, d2, g, u = _prep(degacc, h, alpha[_K].reshape(1, 1))

    for j in range(_K - 1, 0, -1):
        sacc = _sc_gather_scatter(u, srcg, dsts, zeros)
        u = _combine(d2, g, sacc, alpha[j].reshape(1, 1))
    sacc = _sc_gather_scatter(u, srcg, dsts, zeros)
    return _combine(d1, h, sacc, alpha[0].reshape(1, 1))


# speedup vs baseline: 12.7464x; 1.0958x over previous
import functools
from math import comb

import numpy as np
import jax
import jax.numpy as jnp
from jax import lax
from jax.experimental import pallas as pl
from jax.experimental.pallas import tpu as pltpu
from jax.experimental.pallas import tpu_sc as plsc

_K = 10
_N = 10000
_E = 160000
_D = 128

_NPAD = 10112
_GARBAGE = _N
_CHUNK = 64
_NTILES = 32
_CPT = 80
_NBUF = 4
_EPAD = _NTILES * _CPT * _CHUNK
_ROWS_PT = _NPAD // 16
_BLK = 1000
_GRID = _N // _BLK

_GW = np.zeros((_K + 1, _K + 1), np.float32)
for _m in range(_K + 1):
    for _j in range(_K + 1):
        _s = 0
        for _a in range(max(0, _j - (_K - _m)), min(_m, _j) + 1):
            _s += comb(_m, _a) * (-1) ** _a * comb(_K - _m, _j - _a)

        _GW[_m, _j] = comb(_K, _m) * _s


_sc_mesh = plsc.VectorSubcoreMesh(core_axis_name="c", subcore_axis_name="s")


@functools.partial(
    pl.kernel,
    out_type=jax.ShapeDtypeStruct((2, _NPAD, _D), jnp.float32),
    mesh=_sc_mesh,
    scratch_types=[
        pltpu.VMEM((_CPT // 2, _CHUNK), jnp.int32),
        pltpu.VMEM((_CPT // 2, _CHUNK), jnp.int32),
        [pltpu.VMEM((_CHUNK, _D), jnp.float32) for _ in range(_NBUF)],
        pltpu.VMEM_SHARED((_NPAD, _D), jnp.float32),
        [pltpu.SemaphoreType.DMA for _ in range(_NBUF)],
        [pltpu.SemaphoreType.DMA for _ in range(_NBUF)],
    ],
)
def _sc_gather_scatter(table_hbm, srcg_hbm, dsts_hbm, zeros_hbm, out_hbm,
                       src_v, dst_v, bufs, acc, gsems, ssems):
    c = lax.axis_index("c")
    s = lax.axis_index("s")
    wid = c * 16 + s
    pltpu.sync_copy(zeros_hbm.at[pl.ds(s * _ROWS_PT, _ROWS_PT)],
                    acc.at[pl.ds(s * _ROWS_PT, _ROWS_PT)])
    plsc.subcore_barrier()

    half = _CPT // 2

    def _gather(i, b):
        pltpu.async_copy(table_hbm.at[src_v.at[i]], bufs[b], gsems[b])

    def _wait_gather(i, b):
        pltpu.make_async_copy(table_hbm.at[src_v.at[i]], bufs[b],
                              gsems[b]).wait()

    def _scatter(i, b):
        pltpu.async_copy(bufs[b], acc.at[dst_v.at[i]], ssems[b], add=True)

    def _wait_scatter(b):
        pltpu.make_async_copy(bufs[b], acc.at[dst_v.at[0]], ssems[b]).wait()

    for p in range(2):
        pltpu.sync_copy(srcg_hbm.at[pl.ds(wid * _CPT + p * half, half)], src_v)
        pltpu.sync_copy(dsts_hbm.at[pl.ds(wid * _CPT + p * half, half)], dst_v)
        _gather(0, 0)
        _gather(1, 1)
        for i in (0, 1):
            _wait_gather(i, i)
            _scatter(i, i)
            _gather(i + 2, i + 2)

        def body(io, carry):
            i0 = io * _NBUF + 2
            for k in range(_NBUF):
                i = i0 + k
                b = (2 + k) % _NBUF
                _wait_gather(i, b)
                _scatter(i, b)
                _wait_scatter(k)
                _gather(i + 2, k)
            return carry

        lax.fori_loop(0, (half - _NBUF) // _NBUF, body, 0)
        for i in (half - 2, half - 1):
            b = i % _NBUF
            _wait_gather(i, b)
            _scatter(i, b)
        for b in range(_NBUF):
            _wait_scatter(b)
    plsc.subcore_barrier()
    pltpu.sync_copy(acc.at[pl.ds(s * _ROWS_PT, _ROWS_PT)],
                    out_hbm.at[c, pl.ds(s * _ROWS_PT, _ROWS_PT)])


_DW = 16


@functools.partial(
    pl.kernel,
    out_type=jax.ShapeDtypeStruct((2, _NPAD, _DW), jnp.float32),
    mesh=_sc_mesh,
    scratch_types=[
        pltpu.VMEM((_CPT, _CHUNK), jnp.int32),
        pltpu.VMEM((_CHUNK, _DW), jnp.float32),
        pltpu.VMEM_SHARED((_NPAD, _DW), jnp.float32),
    ],
)
def _sc_degree(srcs_hbm, zeros16_hbm, ones16_hbm, out_hbm, src_v, ones_v, acc):
    c = lax.axis_index("c")
    s = lax.axis_index("s")
    wid = c * 16 + s
    pltpu.sync_copy(zeros16_hbm.at[pl.ds(s * _ROWS_PT, _ROWS_PT)],
                    acc.at[pl.ds(s * _ROWS_PT, _ROWS_PT)])
    pltpu.sync_copy(srcs_hbm.at[pl.ds(wid * _CPT, _CPT)], src_v)
    pltpu.sync_copy(ones16_hbm, ones_v)
    plsc.subcore_barrier()

    def body(i, carry):
        pltpu.sync_copy(ones_v, acc.at[src_v.at[i]], add=True)
        return carry

    lax.fori_loop(0, _CPT, body, 0)
    plsc.subcore_barrier()
    pltpu.sync_copy(acc.at[pl.ds(s * _ROWS_PT, _ROWS_PT)],
                    out_hbm.at[c, pl.ds(s * _ROWS_PT, _ROWS_PT)])


def _mlp_body(x_ref, w1t_ref, b1_ref, w2t_ref, b2_ref, o_ref):
    h1 = jnp.dot(x_ref[...], w1t_ref[...],
                 preferred_element_type=jnp.float32,
                 precision=lax.Precision.HIGHEST) + b1_ref[...]
    h1 = jnp.maximum(h1, 0.0)
    o_ref[...] = jnp.dot(h1, w2t_ref[...],
                         preferred_element_type=jnp.float32,
                         precision=lax.Precision.HIGHEST) + b2_ref[...]


def _mlp(x, w1t, b1, w2t, b2):
    return pl.pallas_call(
        _mlp_body,
        grid=(_GRID,),
        in_specs=[
            pl.BlockSpec((_BLK, _D), lambda i: (i, 0)),
            pl.BlockSpec((_D, _D), lambda i: (0, 0)),
            pl.BlockSpec((1, _D), lambda i: (0, 0)),
            pl.BlockSpec((_D, _D), lambda i: (0, 0)),
            pl.BlockSpec((1, _D), lambda i: (0, 0)),
        ],
        out_specs=pl.BlockSpec((_BLK, _D), lambda i: (i, 0)),
        out_shape=jax.ShapeDtypeStruct((_N, _D), jnp.float32),
    )(x, w1t, b1, w2t, b2)


def _prep_body(acc_ref, h_ref, alpha_ref, d1_ref, d2_ref, g_ref, u_ref):
    deg = acc_ref[0][:, 0:1] + acc_ref[1][:, 0:1]
    d1 = jnp.where(deg > 0.0, lax.rsqrt(deg), 0.0)
    d1_ref[...] = d1
    d2_ref[...] = d1 * d1
    g = d1 * h_ref[...]
    g_ref[...] = g
    u_ref[...] = alpha_ref[0, 0] * g


def _prep(degacc, h, alpha_k):
    return pl.pallas_call(
        _prep_body,
        grid=(_GRID,),
        in_specs=[
            pl.BlockSpec((2, _BLK, _DW), lambda i: (0, i, 0)),
            pl.BlockSpec((_BLK, _D), lambda i: (i, 0)),
            pl.BlockSpec(memory_space=pltpu.SMEM),
        ],
        out_specs=[
            pl.BlockSpec((_BLK, 1), lambda i: (i, 0)),
            pl.BlockSpec((_BLK, 1), lambda i: (i, 0)),
            pl.BlockSpec((_BLK, _D), lambda i: (i, 0)),
            pl.BlockSpec((_BLK, _D), lambda i: (i, 0)),
        ],
        out_shape=[
            jax.ShapeDtypeStruct((_N, 1), jnp.float32),
            jax.ShapeDtypeStruct((_N, 1), jnp.float32),
            jax.ShapeDtypeStruct((_N, _D), jnp.float32),
            jax.ShapeDtypeStruct((_N, _D), jnp.float32),
        ],
    )(degacc, h, alpha_k)


def _combine_body(scale_ref, base_ref, acc_ref, alpha_ref, o_ref):
    ssum = acc_ref[0] + acc_ref[1]
    o_ref[...] = scale_ref[...] * ssum + alpha_ref[0, 0] * base_ref[...]


def _combine(scale, base, sacc, alpha_j):
    return pl.pallas_call(
        _combine_body,
        grid=(_GRID,),
        in_specs=[
            pl.BlockSpec((_BLK, 1), lambda i: (i, 0)),
            pl.BlockSpec((_BLK, _D), lambda i: (i, 0)),
            pl.BlockSpec((2, _BLK, _D), lambda i: (0, i, 0)),
            pl.BlockSpec(memory_space=pltpu.SMEM),
        ],
        out_specs=pl.BlockSpec((_BLK, _D), lambda i: (i, 0)),
        out_shape=jax.ShapeDtypeStruct((_N, _D), jnp.float32),
    )(scale, base, sacc, alpha_j)


def kernel(x, edge_index, W1, b1, W2, b2, temp):
    t = jax.nn.relu(temp)
    alpha = jnp.sum(t[:, None] * jnp.asarray(_GW), axis=0) * (1.0 / 2 ** _K)

    h = _mlp(x, W1.T, b1.reshape(1, _D), W2.T, b2.reshape(1, _D))

    src = edge_index[0]
    dst = edge_index[1]
    npad_e = _EPAD - _E
    pad_g = jnp.zeros((npad_e,), jnp.int32)
    pad_s = jnp.full((npad_e,), _GARBAGE, jnp.int32)
    srcg = jnp.concatenate([src, pad_g]).reshape(_EPAD // _CHUNK, _CHUNK)
    dsts = jnp.concatenate([dst, pad_s]).reshape(_EPAD // _CHUNK, _CHUNK)
    srcs = jnp.concatenate([src, pad_s]).reshape(_EPAD // _CHUNK, _CHUNK)

    zeros = jnp.zeros((_NPAD, _D), jnp.float32)
    zeros16 = jnp.zeros((_NPAD, _DW), jnp.float32)
    ones16 = jnp.ones((_CHUNK, _DW), jnp.float32)

    degacc = _sc_degree(srcs, zeros16, ones16)
    d1, d2, g, u = _prep(degacc, h, alpha[_K].reshape(1, 1))

    for j in range(_K - 1, 0, -1):
        sacc = _sc_gather_scatter(u, srcg, dsts, zeros)
        u = _combine(d2, g, sacc, alpha[j].reshape(1, 1))
    sacc = _sc_gather_scatter(u, srcg, dsts, zeros)
    return _combine(d1, h, sacc, alpha[0].reshape(1, 1))
